# baseline (speedup 1.0000x reference)
import functools

import jax
import jax.numpy as jnp
from jax import lax
from jax.experimental import pallas as pl
from jax.experimental.pallas import tpu as pltpu
from jax.experimental.pallas import tpu_sc as plsc

B, S = 4096, 50
NUM_CHAMP, NUM_ITEM, NUM_TRAIT = 60, 60, 27
D_CHAMP, D_ITEM, D_TRAIT = 30, 10, 8
STATS = 31
D_IN = 13 + STATS
D_OUT = 166

_NW = 32
_LW = B // _NW
_L = 16


def _sc_body(x_hbm, champ_hbm, item_hbm, trait_hbm, out_hbm,
             xv, ov, cv, iv, tv):
    wid = lax.axis_index("s") * 2 + lax.axis_index("c")
    base = wid * _LW

    pltpu.sync_copy(champ_hbm, cv)
    pltpu.sync_copy(item_hbm, iv)
    pltpu.sync_copy(trait_hbm, tv)

    def per_s(s, carry):
        pltpu.sync_copy(x_hbm.at[s, :, pl.ds(base, _LW)], xv)

        @plsc.parallel_loop(0, _LW // _L, 1, unroll=8)
        def per_j(j):
            sl = pl.ds(j * _L, _L)

            def lookup(tab_ref, id_row, width, dst_row):
                addr = xv[id_row, sl].astype(jnp.int32) * width
                for e in range(width):
                    ov[dst_row + e, sl] = plsc.load_gather(tab_ref, [addr + e])

            lookup(cv, 0, D_CHAMP, 0)
            lookup(iv, 3, D_ITEM, 30)
            lookup(iv, 4, D_ITEM, 40)
            lookup(iv, 5, D_ITEM, 50)
            for t in range(7):
                lookup(tv, 6 + t, D_TRAIT, 60 + D_TRAIT * t)
            stars = xv[1, sl]
            for v in range(4):
                ov[116 + v, sl] = jnp.where(stars == float(v), 1.0, 0.0)
            cost = xv[2, sl]
            for v in range(15):
                ov[120 + v, sl] = jnp.where(cost == float(v), 1.0, 0.0)
            for c in range(STATS):
                ov[135 + c, sl] = xv[13 + c, sl]

        pltpu.sync_copy(ov, out_hbm.at[s, :, pl.ds(base, _LW)])
        return carry

    lax.fori_loop(0, S, per_s, 0)


@jax.jit
def kernel(x, champ_table, item_table, trait_table):
    xt = jnp.transpose(x, (1, 2, 0))
    run = functools.partial(
        pl.kernel,
        mesh=plsc.VectorSubcoreMesh(core_axis_name="c", subcore_axis_name="s"),
        compiler_params=pltpu.CompilerParams(needs_layout_passes=False),
        out_type=jax.ShapeDtypeStruct((S, D_OUT, B), jnp.float32),
        scratch_types=[
            pltpu.VMEM((D_IN, _LW), jnp.float32),
            pltpu.VMEM((D_OUT, _LW), jnp.float32),
            pltpu.VMEM((NUM_CHAMP * D_CHAMP,), jnp.float32),
            pltpu.VMEM((NUM_ITEM * D_ITEM,), jnp.float32),
            pltpu.VMEM((NUM_TRAIT * D_TRAIT,), jnp.float32),
        ],
    )(_sc_body)
    out_t = run(xt, champ_table.reshape(-1), item_table.reshape(-1),
                trait_table.reshape(-1))
    return jnp.transpose(out_t, (2, 0, 1))


# speedup vs baseline: 1.4384x; 1.4384x over previous
import functools

import jax
import jax.numpy as jnp
from jax import lax
from jax.experimental import pallas as pl
from jax.experimental.pallas import tpu as pltpu
from jax.experimental.pallas import tpu_sc as plsc

B, S = 4096, 50
NUM_CHAMP, NUM_ITEM, NUM_TRAIT = 60, 60, 27
D_CHAMP, D_ITEM, D_TRAIT = 30, 10, 8
STATS = 31
D_IN = 13 + STATS
D_OUT = 166

_NW = 32
_LW = B // _NW
_L = 16


def _compute(xv, ov):
    @plsc.parallel_loop(0, _LW // _L, 1, unroll=2)
    def per_j(j):
        sl = pl.ds(j * _L, _L)

        def lookup(tab_ref, id_row, width, dst_row):
            addr = xv[id_row, sl].astype(jnp.int32) * width
            for e in range(width):
                ov[dst_row + e, sl] = plsc.load_gather(tab_ref, [addr + e])

        lookup(cv_g[0], 0, D_CHAMP, 0)
        lookup(cv_g[1], 3, D_ITEM, 30)
        lookup(cv_g[1], 4, D_ITEM, 40)
        lookup(cv_g[1], 5, D_ITEM, 50)
        for t in range(7):
            lookup(cv_g[2], 6 + t, D_TRAIT, 60 + D_TRAIT * t)
        stars = xv[1, sl]
        for v in range(4):
            ov[116 + v, sl] = jnp.where(stars == float(v), 1.0, 0.0)
        cost = xv[2, sl]
        for v in range(15):
            ov[120 + v, sl] = jnp.where(cost == float(v), 1.0, 0.0)
        for c in range(STATS):
            ov[135 + c, sl] = xv[13 + c, sl]


cv_g = [None, None, None]


def _sc_body(x_hbm, champ_hbm, item_hbm, trait_hbm, out_hbm,
             xv0, xv1, ov0, ov1, cv, iv, tv, si0, si1, so0, so1):
    wid = lax.axis_index("s") * 2 + lax.axis_index("c")
    lanes = pl.ds(wid * _LW, _LW)
    cv_g[0], cv_g[1], cv_g[2] = cv, iv, tv

    pltpu.sync_copy(champ_hbm, cv)
    pltpu.sync_copy(item_hbm, iv)
    pltpu.sync_copy(trait_hbm, tv)

    pltpu.async_copy(x_hbm.at[0, :, lanes], xv0, si0)

    def per_pair(p, carry):
        s0 = 2 * p
        pltpu.async_copy(x_hbm.at[s0 + 1, :, lanes], xv1, si1)
        pltpu.make_async_copy(x_hbm.at[s0, :, lanes], xv0, si0).wait()

        @pl.when(p > 0)
        def _():
            pltpu.make_async_copy(ov0, out_hbm.at[s0 - 2, :, lanes], so0).wait()

        _compute(xv0, ov0)
        pltpu.async_copy(ov0, out_hbm.at[s0, :, lanes], so0)

        @pl.when(p < S // 2 - 1)
        def _():
            pltpu.async_copy(x_hbm.at[s0 + 2, :, lanes], xv0, si0)

        pltpu.make_async_copy(x_hbm.at[s0 + 1, :, lanes], xv1, si1).wait()

        @pl.when(p > 0)
        def _():
            pltpu.make_async_copy(ov1, out_hbm.at[s0 - 1, :, lanes], so1).wait()

        _compute(xv1, ov1)
        pltpu.async_copy(ov1, out_hbm.at[s0 + 1, :, lanes], so1)
        return carry

    lax.fori_loop(0, S // 2, per_pair, 0)
    pltpu.make_async_copy(ov0, out_hbm.at[S - 2, :, lanes], so0).wait()
    pltpu.make_async_copy(ov1, out_hbm.at[S - 1, :, lanes], so1).wait()


@jax.jit
def kernel(x, champ_table, item_table, trait_table):
    xt = jnp.transpose(x, (1, 2, 0))
    run = functools.partial(
        pl.kernel,
        mesh=plsc.VectorSubcoreMesh(core_axis_name="c", subcore_axis_name="s"),
        compiler_params=pltpu.CompilerParams(needs_layout_passes=False),
        out_type=jax.ShapeDtypeStruct((S, D_OUT, B), jnp.float32),
        scratch_types=[
            pltpu.VMEM((D_IN, _LW), jnp.float32),
            pltpu.VMEM((D_IN, _LW), jnp.float32),
            pltpu.VMEM((D_OUT, _LW), jnp.float32),
            pltpu.VMEM((D_OUT, _LW), jnp.float32),
            pltpu.VMEM((NUM_CHAMP * D_CHAMP,), jnp.float32),
            pltpu.VMEM((NUM_ITEM * D_ITEM,), jnp.float32),
            pltpu.VMEM((NUM_TRAIT * D_TRAIT,), jnp.float32),
            pltpu.SemaphoreType.DMA,
            pltpu.SemaphoreType.DMA,
            pltpu.SemaphoreType.DMA,
            pltpu.SemaphoreType.DMA,
        ],
    )(_sc_body)
    out_t = run(xt, champ_table.reshape(-1), item_table.reshape(-1),
                trait_table.reshape(-1))
    return jnp.transpose(out_t, (2, 0, 1))
